# baseline (speedup 1.0000x reference)
import jax
import jax.numpy as jnp
from jax.experimental import pallas as pl


_BLK = 4096


def _copy_body(in_ref, out_ref):
    out_ref[...] = in_ref[...]


def kernel(inputs):
    b, s, h, w, d = inputs.shape
    rows = b * s * h
    x = inputs.reshape(rows, w * d)
    out = pl.pallas_call(
        _copy_body,
        grid=(rows // _BLK,),
        in_specs=[pl.BlockSpec((_BLK, d), lambda i: (i, 1))],
        out_specs=pl.BlockSpec((_BLK, d), lambda i: (i, 0)),
        out_shape=jax.ShapeDtypeStruct((rows, d), jnp.float32),
    )(x)
    return out.reshape(b, s, h, 1, d)


# speedup vs baseline: 5.3020x; 5.3020x over previous
import functools

import jax
import jax.numpy as jnp
from jax import lax
from jax.experimental import pallas as pl
from jax.experimental.pallas import tpu as pltpu
from jax.experimental.pallas import tpu_sc as plsc

_NC = 2
_NS = 16
_NW = _NC * _NS
_ROWS = 2 * 2048 * 16
_RPW = _ROWS // _NW
_CH = 256
_NCHUNK = _RPW // _CH


def _make_sc_copy():
    mesh = plsc.VectorSubcoreMesh(core_axis_name="c", subcore_axis_name="s")

    @functools.partial(
        pl.kernel,
        mesh=mesh,
        out_type=jax.ShapeDtypeStruct((_ROWS, 1, 128), jnp.float32),
        scratch_types=[
            pltpu.VMEM((_CH, 1, 128), jnp.float32),
            pltpu.VMEM((_CH, 1, 128), jnp.float32),
            pltpu.SemaphoreType.DMA,
            pltpu.SemaphoreType.DMA,
        ],
    )
    def sc_copy(in_hbm, out_hbm, buf0, buf1, sem0, sem1):
        wid = lax.axis_index("s") * _NC + lax.axis_index("c")
        base = wid * _RPW
        bufs = (buf0, buf1)
        sems = (sem0, sem1)

        def src(g):
            return in_hbm.at[pl.ds(base + g * _CH, _CH), pl.ds(1, 1)]

        def dst(g):
            return out_hbm.at[pl.ds(base + g * _CH, _CH)]

        pending = pltpu.async_copy(src(0), bufs[0], sems[0])
        for g in range(_NCHUNK):
            nxt = None
            if g + 1 < _NCHUNK:
                nxt = pltpu.async_copy(src(g + 1), bufs[(g + 1) % 2],
                                       sems[(g + 1) % 2])
            pending.wait()
            pltpu.sync_copy(bufs[g % 2], dst(g))
            pending = nxt

    return sc_copy


_sc_copy = _make_sc_copy()


def kernel(inputs):
    b, s, h, w, d = inputs.shape
    x = inputs.reshape(b * s * h, w, d)
    out = _sc_copy(x)
    return out.reshape(b, s, h, 1, d)
